# baseline (speedup 1.0000x reference)
import functools

import jax
import jax.numpy as jnp
from jax import lax
from jax.experimental import pallas as pl
from jax.experimental.pallas import tpu as pltpu
from jax.experimental.pallas import tpu_sc as plsc

_NC = 2
_NS = 16
_NW = _NC * _NS
_L = 16


def _sc_body(nchunks, chunk, b_per_w, a_per_t,
             alpha_hbm, idx_hbm, out_hbm, parts_hbm,
             alpha_v, idx_v0, idx_v1, rows_v0, rows_v1, parts_v, parts_all_v,
             sem_i0, sem_i1, sem_g0, sem_g1, sem_o0, sem_o1):
    cid = lax.axis_index("c")
    sid = lax.axis_index("s")
    wid = sid * _NC + cid
    base = wid * b_per_w
    idx_v = [idx_v0, idx_v1]
    rows_v = [rows_v0, rows_v1]
    sem_i = [sem_i0, sem_i1]
    sem_g = [sem_g0, sem_g1]
    sem_o = [sem_o0, sem_o1]

    cp_i = [None] * nchunks
    cp_i[0] = pltpu.async_copy(idx_hbm.at[pl.ds(base, chunk)], idx_v0, sem_i0)
    if nchunks > 1:
        cp_i[1] = pltpu.async_copy(idx_hbm.at[pl.ds(base + chunk, chunk)],
                                   idx_v1, sem_i1)

    pltpu.sync_copy(alpha_hbm.at[pl.ds(sid * a_per_t, a_per_t)], alpha_v)
    nvec = a_per_t // _L

    def _max_body(i, m):
        return jnp.maximum(m, alpha_v[pl.ds(i * _L, _L)])

    m_loc = lax.fori_loop(0, nvec, _max_body,
                          jnp.full((_L,), -jnp.inf, jnp.float32))

    def _sum_body(i, s):
        return s + jnp.exp(alpha_v[pl.ds(i * _L, _L)] - m_loc)

    s_loc = lax.fori_loop(0, nvec, _sum_body, jnp.zeros((_L,), jnp.float32))

    parts_v[0, :] = m_loc
    parts_v[1, :] = s_loc
    pltpu.sync_copy(parts_v, parts_hbm.at[cid * _NS + sid])
    plsc.subcore_barrier()

    cp_g = [None, None]
    pltpu.sync_copy(parts_hbm.at[pl.ds(cid * _NS, _NS)], parts_all_v)
    m_vec = jnp.full((_L,), -jnp.inf, jnp.float32)
    for t in range(_NS):
        m_vec = jnp.maximum(m_vec, parts_all_v[t, 0, :])
    s_vec = jnp.zeros((_L,), jnp.float32)
    for t in range(_NS):
        s_vec = s_vec + (jnp.exp(parts_all_v[t, 0, :] - m_vec)
                         * parts_all_v[t, 1, :])

    m_scl = m_vec[0]
    for j in range(1, _L):
        m_scl = jnp.maximum(m_scl, m_vec[j])
    m_g = jnp.full((_L,), m_scl, jnp.float32)
    w_vec = jnp.exp(m_vec - m_g) * s_vec
    s_scl = w_vec[0]
    for j in range(1, _L):
        s_scl = s_scl + w_vec[j]
    inv = jnp.float32(1.0) / jnp.full((_L,), s_scl, jnp.float32)

    cp_i[0].wait()
    cp_g[0] = pltpu.async_copy(alpha_hbm.at[idx_v0], rows_v0, sem_g0)

    nrm = chunk // _L
    cp_o = [None, None]

    for ch in range(nchunks):
        b = ch & 1
        nb = 1 - b
        if ch + 1 < nchunks:
            if cp_o[nb] is not None:
                cp_o[nb].wait()
            cp_i[ch + 1].wait()
            cp_g[nb] = pltpu.async_copy(alpha_hbm.at[idx_v[nb]], rows_v[nb],
                                        sem_g[nb])
        cp_g[b].wait()
        if ch + 2 < nchunks:
            off = base + (ch + 2) * chunk
            cp_i[ch + 2] = pltpu.async_copy(idx_hbm.at[pl.ds(off, chunk)],
                                            idx_v[b], sem_i[b])
        rv = rows_v[b]

        def _nrm_body(i, _, rv=rv):
            g = rv[pl.ds(i * _L, _L)]
            rv[pl.ds(i * _L, _L)] = jnp.exp(g - m_g) * inv
            return 0

        lax.fori_loop(0, nrm, _nrm_body, 0)
        cp_o[b] = pltpu.async_copy(rv, out_hbm.at[pl.ds(base + ch * chunk,
                                                        chunk)], sem_o[b])
    for cp in cp_o:
        if cp is not None:
            cp.wait()


def kernel(idx, alpha):
    batch, hist = idx.shape
    n = alpha.shape[0]

    bflat = batch * hist
    assert bflat % _NW == 0
    b_per_w = bflat // _NW
    chunk = b_per_w
    nchunks = 1
    while chunk * 16 > 224 * 1024 or chunk % _L != 0:
        nchunks += 1
        while b_per_w % nchunks != 0:
            nchunks += 1
        chunk = b_per_w // nchunks

    a_per_t = -(-n // (_NS * _L)) * _L
    n_pad = _NS * a_per_t - n
    ap = jnp.pad(alpha, (0, n_pad), constant_values=-jnp.inf)

    mesh = plsc.VectorSubcoreMesh(core_axis_name="c", subcore_axis_name="s")
    run = pl.kernel(
        functools.partial(_sc_body, nchunks, chunk, b_per_w, a_per_t),
        out_type=(jax.ShapeDtypeStruct((bflat,), jnp.float32),
                  jax.ShapeDtypeStruct((_NW, 2, _L), jnp.float32)),
        mesh=mesh,
        scratch_types=[
            pltpu.VMEM((a_per_t,), jnp.float32),
            pltpu.VMEM((chunk,), jnp.int32),
            pltpu.VMEM((chunk,), jnp.int32),
            pltpu.VMEM((chunk,), jnp.float32),
            pltpu.VMEM((chunk,), jnp.float32),
            pltpu.VMEM((2, _L), jnp.float32),
            pltpu.VMEM((_NS, 2, _L), jnp.float32),
            pltpu.SemaphoreType.DMA,
            pltpu.SemaphoreType.DMA,
            pltpu.SemaphoreType.DMA,
            pltpu.SemaphoreType.DMA,
            pltpu.SemaphoreType.DMA,
            pltpu.SemaphoreType.DMA,
        ],
    )
    out_flat, _ = run(ap, idx.reshape(-1))
    return out_flat.reshape(batch, hist)


# speedup vs baseline: 1.2363x; 1.2363x over previous
import functools

import jax
import jax.numpy as jnp
from jax import lax
from jax.experimental import pallas as pl
from jax.experimental.pallas import tpu as pltpu
from jax.experimental.pallas import tpu_sc as plsc

_NC = 2
_NS = 16
_NW = _NC * _NS
_L = 16


def _softmax_body(alpha_ref, out_ref):
    a = alpha_ref[...]
    m = jnp.max(a)
    e = jnp.exp(a - m)
    out_ref[...] = e / jnp.sum(e)


def _softmax_table(alpha_padded_2d):
    return pl.pallas_call(
        _softmax_body,
        out_shape=jax.ShapeDtypeStruct(alpha_padded_2d.shape, jnp.float32),
    )(alpha_padded_2d)


def _sc_gather_body(nchunks, chunk, b_per_w, table_hbm, idx_hbm, out_hbm,
                    idx_v0, idx_v1, rows_v0, rows_v1,
                    sem_i0, sem_i1, sem_g0, sem_g1, sem_o0, sem_o1):
    wid = lax.axis_index("s") * _NC + lax.axis_index("c")
    base = wid * b_per_w
    idx_v = [idx_v0, idx_v1]
    rows_v = [rows_v0, rows_v1]
    sem_i = [sem_i0, sem_i1]
    sem_g = [sem_g0, sem_g1]
    sem_o = [sem_o0, sem_o1]

    cp_i = [None] * nchunks
    cp_g = [None, None]
    cp_o = [None, None]
    cp_i[0] = pltpu.async_copy(idx_hbm.at[pl.ds(base, chunk)], idx_v0, sem_i0)
    if nchunks > 1:
        cp_i[1] = pltpu.async_copy(idx_hbm.at[pl.ds(base + chunk, chunk)],
                                   idx_v1, sem_i1)
    cp_i[0].wait()
    cp_g[0] = pltpu.async_copy(table_hbm.at[idx_v0], rows_v0, sem_g0)

    for ch in range(nchunks):
        b = ch & 1
        nb = 1 - b
        if ch + 1 < nchunks:
            if cp_o[nb] is not None:
                cp_o[nb].wait()
            cp_i[ch + 1].wait()
            cp_g[nb] = pltpu.async_copy(table_hbm.at[idx_v[nb]], rows_v[nb],
                                        sem_g[nb])
        cp_g[b].wait()
        if ch + 2 < nchunks:
            off = base + (ch + 2) * chunk
            cp_i[ch + 2] = pltpu.async_copy(idx_hbm.at[pl.ds(off, chunk)],
                                            idx_v[b], sem_i[b])
        cp_o[b] = pltpu.async_copy(
            rows_v[b], out_hbm.at[pl.ds(base + ch * chunk, chunk)], sem_o[b])
    for cp in cp_o:
        if cp is not None:
            cp.wait()


def kernel(idx, alpha):
    batch, hist = idx.shape
    n = alpha.shape[0]

    n_pad = (-n) % 128
    ap = jnp.pad(alpha, (0, n_pad), constant_values=-jnp.inf)
    table = _softmax_table(ap.reshape(-1, 128)).reshape(-1)

    bflat = batch * hist
    assert bflat % (8 * _NW) == 0
    b_per_w = bflat // _NW
    chunk = b_per_w
    nchunks = 1
    while chunk * 16 > 448 * 1024 or chunk % _L != 0:
        nchunks += 1
        while b_per_w % nchunks != 0:
            nchunks += 1
        chunk = b_per_w // nchunks

    mesh = plsc.VectorSubcoreMesh(core_axis_name="c", subcore_axis_name="s")
    gather = pl.kernel(
        functools.partial(_sc_gather_body, nchunks, chunk, b_per_w),
        out_type=jax.ShapeDtypeStruct((bflat,), jnp.float32),
        mesh=mesh,
        scratch_types=[
            pltpu.VMEM((chunk,), jnp.int32),
            pltpu.VMEM((chunk,), jnp.int32),
            pltpu.VMEM((chunk,), jnp.float32),
            pltpu.VMEM((chunk,), jnp.float32),
            pltpu.SemaphoreType.DMA,
            pltpu.SemaphoreType.DMA,
            pltpu.SemaphoreType.DMA,
            pltpu.SemaphoreType.DMA,
            pltpu.SemaphoreType.DMA,
            pltpu.SemaphoreType.DMA,
        ],
    )
    out_flat = gather(table, idx.reshape(-1))
    return out_flat.reshape(batch, hist)


# speedup vs baseline: 1.9889x; 1.6088x over previous
import functools

import jax
import jax.numpy as jnp
from jax import lax
from jax.experimental import pallas as pl
from jax.experimental.pallas import tpu as pltpu
from jax.experimental.pallas import tpu_sc as plsc

_NC = 2
_NS = 16
_NW = _NC * _NS
_L = 16


def _softmax_body(alpha_ref, out_ref):
    a = alpha_ref[...]
    m = jnp.max(a)
    e = jnp.exp(a - m)
    out_ref[...] = e / jnp.sum(e)


def _softmax_table(alpha_padded_2d):
    return pl.pallas_call(
        _softmax_body,
        out_shape=jax.ShapeDtypeStruct(alpha_padded_2d.shape, jnp.float32),
    )(alpha_padded_2d)


def _sc_gather_body(nchunks, chunk, b_per_w, t_per_t, table_hbm, idx_hbm,
                    out_hbm, idx_v0, idx_v1, rows_v0, rows_v1, shared,
                    sem_i0, sem_i1, sem_g0, sem_g1, sem_o0, sem_o1):
    sid = lax.axis_index("s")
    wid = sid * _NC + lax.axis_index("c")
    base = wid * b_per_w
    idx_v = [idx_v0, idx_v1]
    rows_v = [rows_v0, rows_v1]
    sem_i = [sem_i0, sem_i1]
    sem_g = [sem_g0, sem_g1]
    sem_o = [sem_o0, sem_o1]

    cp_i = [None] * nchunks
    cp_g = [None, None]
    cp_o = [None, None]
    cp_i[0] = pltpu.async_copy(idx_hbm.at[pl.ds(base, chunk)], idx_v0, sem_i0)
    if nchunks > 1:
        cp_i[1] = pltpu.async_copy(idx_hbm.at[pl.ds(base + chunk, chunk)],
                                   idx_v1, sem_i1)

    toff = sid * t_per_t
    done = 0
    while done < t_per_t:
        piece = min(chunk, t_per_t - done)
        pltpu.sync_copy(table_hbm.at[pl.ds(toff + done, piece)],
                        rows_v0.at[pl.ds(0, piece)])
        pltpu.sync_copy(rows_v0.at[pl.ds(0, piece)],
                        shared.at[pl.ds(toff + done, piece)])
        done += piece
    plsc.subcore_barrier()

    cp_i[0].wait()
    cp_g[0] = pltpu.async_copy(shared.at[idx_v0], rows_v0, sem_g0)

    for ch in range(nchunks):
        b = ch & 1
        nb = 1 - b
        if ch + 1 < nchunks:
            if cp_o[nb] is not None:
                cp_o[nb].wait()
            cp_i[ch + 1].wait()
            cp_g[nb] = pltpu.async_copy(shared.at[idx_v[nb]], rows_v[nb],
                                        sem_g[nb])
        cp_g[b].wait()
        if ch + 2 < nchunks:
            off = base + (ch + 2) * chunk
            cp_i[ch + 2] = pltpu.async_copy(idx_hbm.at[pl.ds(off, chunk)],
                                            idx_v[b], sem_i[b])
        cp_o[b] = pltpu.async_copy(
            rows_v[b], out_hbm.at[pl.ds(base + ch * chunk, chunk)], sem_o[b])
    for cp in cp_o:
        if cp is not None:
            cp.wait()


def kernel(idx, alpha):
    batch, hist = idx.shape
    n = alpha.shape[0]

    n_pad = (-n) % 128
    ap = jnp.pad(alpha, (0, n_pad), constant_values=-jnp.inf)
    table = _softmax_table(ap.reshape(-1, 128)).reshape(-1)

    bflat = batch * hist
    assert bflat % (8 * _NW) == 0
    b_per_w = bflat // _NW
    chunk = b_per_w
    nchunks = 1
    while chunk * 16 > 208 * 1024 or chunk % _L != 0:
        nchunks += 1
        while b_per_w % nchunks != 0:
            nchunks += 1
        chunk = b_per_w // nchunks

    n_table = n + n_pad
    assert n_table % (8 * _NS) == 0
    t_per_t = n_table // _NS

    mesh = plsc.VectorSubcoreMesh(core_axis_name="c", subcore_axis_name="s")
    gather = pl.kernel(
        functools.partial(_sc_gather_body, nchunks, chunk, b_per_w, t_per_t),
        out_type=jax.ShapeDtypeStruct((bflat,), jnp.float32),
        mesh=mesh,
        scratch_types=[
            pltpu.VMEM((chunk,), jnp.int32),
            pltpu.VMEM((chunk,), jnp.int32),
            pltpu.VMEM((chunk,), jnp.float32),
            pltpu.VMEM((chunk,), jnp.float32),
            pltpu.VMEM_SHARED((n_table,), jnp.float32),
            pltpu.SemaphoreType.DMA,
            pltpu.SemaphoreType.DMA,
            pltpu.SemaphoreType.DMA,
            pltpu.SemaphoreType.DMA,
            pltpu.SemaphoreType.DMA,
            pltpu.SemaphoreType.DMA,
        ],
    )
    out_flat = gather(table, idx.reshape(-1))
    return out_flat.reshape(batch, hist)
